# baseline (speedup 1.0000x reference)
import functools

import jax
import jax.numpy as jnp
from jax import lax
from jax.experimental import pallas as pl
from jax.experimental.pallas import tpu as pltpu
from jax.experimental.pallas import tpu_sc as plsc

NC = 2
NS = 16
NW = NC * NS
CHUNK = 128
D = 300
DPAD = 384
LANES = 16


def _body(table_hbm, idx_hbm, out_hbm, idx_v, rows_v, pack_v, sem):
    nch = idx_hbm.shape[1]
    wid = lax.axis_index("s") * NC + lax.axis_index("c")
    pltpu.sync_copy(idx_hbm.at[wid], idx_v)

    tail_idx = jax.lax.iota(jnp.int32, LANES) + (D // LANES) * LANES
    tail_mask = jax.lax.iota(jnp.int32, LANES) < (D % LANES)

    def repack_row(r, carry):
        for k in range(D // LANES):
            pack_v[r, pl.ds(k * LANES, LANES)] = rows_v[r, pl.ds(k * LANES, LANES)]
        tail = rows_v[r, pl.ds((D // LANES) * LANES, LANES)]
        row_idx = jnp.full((LANES,), r, dtype=jnp.int32)
        plsc.store_scatter(pack_v, [row_idx, tail_idx], tail, mask=tail_mask)
        return carry

    def step(j, carry):
        pltpu.async_copy(table_hbm.at[idx_v.at[j]], rows_v, sem).wait()
        lax.fori_loop(0, CHUNK, repack_row, 0)
        pltpu.sync_copy(pack_v, out_hbm.at[wid * nch + j])
        return carry

    lax.fori_loop(0, nch, step, 0)


@functools.lru_cache(maxsize=None)
def _make(nch):
    mesh = plsc.VectorSubcoreMesh(core_axis_name="c", subcore_axis_name="s")
    return pl.kernel(
        _body,
        out_type=jax.ShapeDtypeStruct((NW * nch, CHUNK, D), jnp.float32),
        mesh=mesh,
        scratch_types=[
            pltpu.VMEM((nch, CHUNK), jnp.int32),
            pltpu.VMEM((CHUNK, DPAD), jnp.float32),
            pltpu.VMEM((CHUNK, D), jnp.float32),
            pltpu.SemaphoreType.DMA,
        ],
        compiler_params=pltpu.CompilerParams(needs_layout_passes=False),
    )


def kernel(word_input, table):
    b, s = word_input.shape
    vocab, d = table.shape
    idx = word_input.reshape(-1).astype(jnp.int32)
    total = b * s
    per_w = total // NW
    nch = per_w // CHUNK
    idx3 = idx.reshape(NW, nch, CHUNK)
    tab_pad = jnp.pad(table, ((0, 0), (0, DPAD - d)))
    out = _make(nch)(tab_pad, idx3)
    return out.reshape(b, s, d)
